# baseline (speedup 1.0000x reference)
import jax
import jax.numpy as jnp
from jax import lax
from jax.experimental import pallas as pl
from jax.experimental.pallas import tpu as pltpu
from jax.experimental.pallas import tpu_sc as plsc

_B = 16384
_D = 64
_LANES = 16
_NCHUNK = _D // _LANES


def _score_body(nodes_hbm, rel_hbm, s_hbm, o_hbm, r_hbm, out_hbm,
                sidx_v, oidx_v, ridx_v,
                rbuf0_v, sbuf0_v, obuf0_v, rbuf1_v, sbuf1_v, obuf1_v,
                out_v, sem0, sem1):
    info = plsc.get_sparse_core_info()
    nw = info.num_cores * info.num_subcores
    bpw = _B // nw
    quart = bpw // 4
    ngrp = quart // _LANES
    wid = lax.axis_index("s") * info.num_cores + lax.axis_index("c")
    base = wid * bpw

    c1 = pltpu.async_copy(s_hbm.at[pl.ds(base, bpw)], sidx_v, sem0)
    c2 = pltpu.async_copy(o_hbm.at[pl.ds(base, bpw)], oidx_v, sem0)
    c3 = pltpu.async_copy(r_hbm.at[pl.ds(base, bpw)], ridx_v, sem0)
    c1.wait()
    c2.wait()
    c3.wait()

    lanes = lax.iota(jnp.int32, _LANES)
    bufs = ((rbuf0_v, sbuf0_v, obuf0_v, sem0),
            (rbuf1_v, sbuf1_v, obuf1_v, sem1))

    def enqueue(q, rbuf_v, sbuf_v, obuf_v, sem):
        def enq(g, c):
            row0 = g * _LANES
            rchunk = ridx_v[pl.ds(q * quart + row0, _LANES)]
            schunk = sidx_v[pl.ds(q * quart + row0, _LANES)]
            ochunk = oidx_v[pl.ds(q * quart + row0, _LANES)]
            for l in range(_LANES):
                pltpu.async_copy(rel_hbm.at[rchunk[l]],
                                 rbuf_v.at[row0 + l], sem)
                pltpu.async_copy(nodes_hbm.at[schunk[l]],
                                 sbuf_v.at[row0 + l], sem)
                pltpu.async_copy(nodes_hbm.at[ochunk[l]],
                                 obuf_v.at[row0 + l], sem)
            return c
        lax.fori_loop(0, ngrp, enq, 0)

    def finish(q, rbuf_v, sbuf_v, obuf_v, sem):
        pltpu.make_async_copy(rel_hbm.at[pl.ds(0, quart)], rbuf_v, sem).wait()
        pltpu.make_async_copy(nodes_hbm.at[pl.ds(0, quart)], sbuf_v,
                              sem).wait()
        pltpu.make_async_copy(nodes_hbm.at[pl.ds(0, quart)], obuf_v,
                              sem).wait()

        def compute(g, c):
            row0 = g * _LANES
            tot = jnp.zeros((_LANES,), jnp.float32)
            for l in range(_LANES):
                sl = pl.ds(0, _LANES)
                acc = (rbuf_v[row0 + l, sl] * sbuf_v[row0 + l, sl]
                       * obuf_v[row0 + l, sl])
                for j in range(1, _NCHUNK):
                    sl = pl.ds(j * _LANES, _LANES)
                    acc = acc + (rbuf_v[row0 + l, sl] * sbuf_v[row0 + l, sl]
                                 * obuf_v[row0 + l, sl])
                tot = jnp.where(lanes == l, jnp.sum(acc), tot)
            out_v[pl.ds(q * quart + row0, _LANES)] = tot
            return c
        lax.fori_loop(0, ngrp, compute, 0)

    enqueue(0, *bufs[0])
    for q in range(1, 4):
        enqueue(q, *bufs[q % 2])
        finish(q - 1, *bufs[(q - 1) % 2])
    finish(3, *bufs[1])

    pltpu.sync_copy(out_v, out_hbm.at[pl.ds(base, bpw)])


def kernel(node_embeddings, s, o, r, rel_embedding):
    info = plsc.get_sparse_core_info()
    nw = info.num_cores * info.num_subcores
    bpw = _B // nw
    mesh = plsc.VectorSubcoreMesh(core_axis_name="c", subcore_axis_name="s")
    run = pl.kernel(
        _score_body,
        out_type=jax.ShapeDtypeStruct((_B,), jnp.float32),
        mesh=mesh,
        compiler_params=pltpu.CompilerParams(needs_layout_passes=False,
                                             use_tc_tiling_on_sc=True),
        scratch_types=[
            pltpu.VMEM((bpw,), jnp.int32),
            pltpu.VMEM((bpw,), jnp.int32),
            pltpu.VMEM((bpw,), jnp.int32),
            pltpu.VMEM((bpw // 4, _D), jnp.float32),
            pltpu.VMEM((bpw // 4, _D), jnp.float32),
            pltpu.VMEM((bpw // 4, _D), jnp.float32),
            pltpu.VMEM((bpw // 4, _D), jnp.float32),
            pltpu.VMEM((bpw // 4, _D), jnp.float32),
            pltpu.VMEM((bpw // 4, _D), jnp.float32),
            pltpu.VMEM((bpw,), jnp.float32),
            pltpu.SemaphoreType.DMA,
            pltpu.SemaphoreType.DMA,
        ],
    )
    nodes_rm = node_embeddings.at[jnp.zeros((8,), jnp.int32)].add(
        jnp.zeros((8, _D), jnp.float32))
    return run(nodes_rm, rel_embedding,
               s.astype(jnp.int32), o.astype(jnp.int32), r.astype(jnp.int32))


# speedup vs baseline: 1.0160x; 1.0160x over previous
import jax
import jax.numpy as jnp
from jax import lax
from jax.experimental import pallas as pl
from jax.experimental.pallas import tpu as pltpu
from jax.experimental.pallas import tpu_sc as plsc

_B = 16384
_D = 64
_LANES = 16
_NCHUNK = _D // _LANES


def _score_body(nodes_hbm, rel_hbm, s_hbm, o_hbm, r_hbm, out_hbm,
                sidx_v, oidx_v, ridx_v, rbuf_v, sbuf_v, obuf_v, out_v, sem):
    info = plsc.get_sparse_core_info()
    nw = info.num_cores * info.num_subcores
    bpw = _B // nw
    half = bpw // 2
    ngrp = half // _LANES
    wid = lax.axis_index("s") * info.num_cores + lax.axis_index("c")
    base = wid * bpw

    c1 = pltpu.async_copy(s_hbm.at[pl.ds(base, bpw)], sidx_v, sem)
    c2 = pltpu.async_copy(o_hbm.at[pl.ds(base, bpw)], oidx_v, sem)
    c3 = pltpu.async_copy(r_hbm.at[pl.ds(base, bpw)], ridx_v, sem)
    c1.wait()
    c2.wait()
    c3.wait()

    lanes = lax.iota(jnp.int32, _LANES)

    for h in range(2):
        def enq(g, c):
            row0 = g * _LANES
            rchunk = ridx_v[pl.ds(h * half + row0, _LANES)]
            schunk = sidx_v[pl.ds(h * half + row0, _LANES)]
            ochunk = oidx_v[pl.ds(h * half + row0, _LANES)]
            for l in range(_LANES):
                pltpu.async_copy(rel_hbm.at[rchunk[l]],
                                 rbuf_v.at[row0 + l], sem)
                pltpu.async_copy(nodes_hbm.at[schunk[l]],
                                 sbuf_v.at[row0 + l], sem)
                pltpu.async_copy(nodes_hbm.at[ochunk[l]],
                                 obuf_v.at[row0 + l], sem)
            return c
        lax.fori_loop(0, ngrp, enq, 0)

        pltpu.make_async_copy(rel_hbm.at[pl.ds(0, half)], rbuf_v, sem).wait()
        pltpu.make_async_copy(nodes_hbm.at[pl.ds(0, half)], sbuf_v,
                              sem).wait()
        pltpu.make_async_copy(nodes_hbm.at[pl.ds(0, half)], obuf_v,
                              sem).wait()

        def compute(g, c):
            row0 = g * _LANES
            tot = jnp.zeros((_LANES,), jnp.float32)
            for l in range(_LANES):
                sl = pl.ds(0, _LANES)
                acc = (rbuf_v[row0 + l, sl] * sbuf_v[row0 + l, sl]
                       * obuf_v[row0 + l, sl])
                for j in range(1, _NCHUNK):
                    sl = pl.ds(j * _LANES, _LANES)
                    acc = acc + (rbuf_v[row0 + l, sl] * sbuf_v[row0 + l, sl]
                                 * obuf_v[row0 + l, sl])
                tot = jnp.where(lanes == l, jnp.sum(acc), tot)
            out_v[pl.ds(h * half + row0, _LANES)] = tot
            return c
        lax.fori_loop(0, ngrp, compute, 0)

    pltpu.sync_copy(out_v, out_hbm.at[pl.ds(base, bpw)])


def kernel(node_embeddings, s, o, r, rel_embedding):
    info = plsc.get_sparse_core_info()
    nw = info.num_cores * info.num_subcores
    bpw = _B // nw
    mesh = plsc.VectorSubcoreMesh(core_axis_name="c", subcore_axis_name="s")
    run = pl.kernel(
        _score_body,
        out_type=jax.ShapeDtypeStruct((_B,), jnp.float32),
        mesh=mesh,
        compiler_params=pltpu.CompilerParams(needs_layout_passes=False,
                                             use_tc_tiling_on_sc=True),
        scratch_types=[
            pltpu.VMEM((bpw,), jnp.int32),
            pltpu.VMEM((bpw,), jnp.int32),
            pltpu.VMEM((bpw,), jnp.int32),
            pltpu.VMEM((bpw // 2, _D), jnp.float32),
            pltpu.VMEM((bpw // 2, _D), jnp.float32),
            pltpu.VMEM((bpw // 2, _D), jnp.float32),
            pltpu.VMEM((bpw,), jnp.float32),
            pltpu.SemaphoreType.DMA,
        ],
    )
    nodes_rm = node_embeddings.at[jnp.zeros((8,), jnp.int32)].add(
        jnp.zeros((8, _D), jnp.float32))
    return run(nodes_rm, rel_embedding,
               s.astype(jnp.int32), o.astype(jnp.int32), r.astype(jnp.int32))
